# Initial kernel scaffold; baseline (speedup 1.0000x reference)
import jax
import jax.numpy as jnp
from jax.experimental import pallas as pl


def kernel(outputs, targets):
    raise NotImplementedError("write your pallas kernel here")


# speedup vs baseline: 4.3745x; 4.3745x over previous
import functools

import jax
import jax.numpy as jnp
from jax.experimental import pallas as pl

_ROWS_PER_BLOCK = 8


def _acc_kernel(x_ref, t_ref, acc1_ref, acc5_ref, *, scale):
    i = pl.program_id(0)
    x = x_ref[...]
    t = t_ref[...]
    r, n = x.shape
    col = jax.lax.broadcasted_iota(jnp.int32, (r, n), 1)
    vt = jnp.max(jnp.where(col == t, x, -jnp.inf), axis=1, keepdims=True)
    gt = jnp.sum((x > vt).astype(jnp.int32), axis=1, keepdims=True)
    eq_before = jnp.sum(((x == vt) & (col < t)).astype(jnp.int32),
                        axis=1, keepdims=True)
    rank = gt + eq_before
    a1 = jnp.sum((rank < 1).astype(jnp.float32)).reshape(1, 1) * scale
    a5 = jnp.sum((rank < 5).astype(jnp.float32)).reshape(1, 1) * scale

    @pl.when(i == 0)
    def _init():
        acc1_ref[...] = a1
        acc5_ref[...] = a5

    @pl.when(i != 0)
    def _accum():
        acc1_ref[...] += a1
        acc5_ref[...] += a5


@jax.jit
def kernel(outputs, targets):
    b, n = outputs.shape
    r = _ROWS_PER_BLOCK
    t2 = targets.astype(jnp.int32).reshape(b, 1)
    body = functools.partial(_acc_kernel, scale=100.0 / b)
    a1, a5 = pl.pallas_call(
        body,
        grid=(b // r,),
        in_specs=[
            pl.BlockSpec((r, n), lambda i: (i, 0)),
            pl.BlockSpec((r, 1), lambda i: (i, 0)),
        ],
        out_specs=[
            pl.BlockSpec((1, 1), lambda i: (0, 0)),
            pl.BlockSpec((1, 1), lambda i: (0, 0)),
        ],
        out_shape=[
            jax.ShapeDtypeStruct((1, 1), jnp.float32),
            jax.ShapeDtypeStruct((1, 1), jnp.float32),
        ],
    )(outputs, t2)
    return (a1.reshape(1), a5.reshape(1))


# speedup vs baseline: 4.8795x; 1.1154x over previous
import functools

import jax
import jax.numpy as jnp
from jax.experimental import pallas as pl

_ROWS_PER_BLOCK = 8


def _acc_kernel(x_ref, t_ref, acc1_ref, acc5_ref, *, scale, nchunks):
    i = pl.program_id(0)
    t = t_ref[...]
    r, n = x_ref.shape
    cw = n // nchunks
    maxes = []
    for c in range(nchunks):
        xc = x_ref[:, c * cw:(c + 1) * cw]
        colc = jax.lax.broadcasted_iota(jnp.int32, (r, cw), 1) + c * cw
        maxes.append(jnp.max(jnp.where(colc == t, xc, -jnp.inf),
                             axis=1, keepdims=True))
    vt = functools.reduce(jnp.maximum, maxes)
    cnts = []
    for c in range(nchunks):
        xc = x_ref[:, c * cw:(c + 1) * cw]
        colc = jax.lax.broadcasted_iota(jnp.int32, (r, cw), 1) + c * cw
        pred = (xc > vt) | ((xc == vt) & (colc < t))
        cnts.append(jnp.sum(pred.astype(jnp.float32), axis=1, keepdims=True))
    rank = functools.reduce(jnp.add, cnts)
    a1 = jnp.sum((rank < 1.0).astype(jnp.float32)).reshape(1, 1) * scale
    a5 = jnp.sum((rank < 5.0).astype(jnp.float32)).reshape(1, 1) * scale

    @pl.when(i == 0)
    def _init():
        acc1_ref[...] = a1
        acc5_ref[...] = a5

    @pl.when(i != 0)
    def _accum():
        acc1_ref[...] += a1
        acc5_ref[...] += a5


@jax.jit
def kernel(outputs, targets):
    b, n = outputs.shape
    r = _ROWS_PER_BLOCK
    t2 = targets.astype(jnp.int32).reshape(b, 1)
    body = functools.partial(_acc_kernel, scale=100.0 / b, nchunks=8)
    a1, a5 = pl.pallas_call(
        body,
        grid=(b // r,),
        in_specs=[
            pl.BlockSpec((r, n), lambda i: (i, 0)),
            pl.BlockSpec((r, 1), lambda i: (i, 0)),
        ],
        out_specs=[
            pl.BlockSpec((1, 1), lambda i: (0, 0)),
            pl.BlockSpec((1, 1), lambda i: (0, 0)),
        ],
        out_shape=[
            jax.ShapeDtypeStruct((1, 1), jnp.float32),
            jax.ShapeDtypeStruct((1, 1), jnp.float32),
        ],
    )(outputs, t2)
    return (a1.reshape(1), a5.reshape(1))


# speedup vs baseline: 6.5322x; 1.3387x over previous
import functools

import jax
import jax.numpy as jnp
from jax.experimental import pallas as pl

_ROWS_PER_BLOCK = 16
_NCHUNKS = 8


def _acc_kernel(x_ref, t_ref, acc1_ref, acc5_ref, *, scale, nchunks):
    i = pl.program_id(0)
    r, n = x_ref.shape
    cw = n // nchunks
    t = t_ref[pl.ds(i * r, r), :]
    maxes = []
    for c in range(nchunks):
        xc = x_ref[:, c * cw:(c + 1) * cw]
        colc = jax.lax.broadcasted_iota(jnp.int32, (r, cw), 1) + c * cw
        maxes.append(jnp.max(jnp.where(colc == t, xc, -jnp.inf),
                             axis=1, keepdims=True))
    vt = functools.reduce(jnp.maximum, maxes)
    cnts = []
    for c in range(nchunks):
        xc = x_ref[:, c * cw:(c + 1) * cw]
        colc = jax.lax.broadcasted_iota(jnp.int32, (r, cw), 1) + c * cw
        pred = (xc > vt) | ((xc == vt) & (colc < t))
        cnts.append(jnp.sum(pred.astype(jnp.float32), axis=1, keepdims=True))
    rank = functools.reduce(jnp.add, cnts)
    a1 = jnp.sum((rank < 1.0).astype(jnp.float32)).reshape(1, 1) * scale
    a5 = jnp.sum((rank < 5.0).astype(jnp.float32)).reshape(1, 1) * scale

    @pl.when(i == 0)
    def _init():
        acc1_ref[...] = a1
        acc5_ref[...] = a5

    @pl.when(i != 0)
    def _accum():
        acc1_ref[...] += a1
        acc5_ref[...] += a5


@jax.jit
def kernel(outputs, targets):
    b, n = outputs.shape
    r = _ROWS_PER_BLOCK
    t2 = targets.astype(jnp.int32).reshape(b, 1)
    body = functools.partial(_acc_kernel, scale=100.0 / b, nchunks=_NCHUNKS)
    a1, a5 = pl.pallas_call(
        body,
        grid=(b // r,),
        in_specs=[
            pl.BlockSpec((r, n), lambda i: (i, 0)),
            pl.BlockSpec((b, 1), lambda i: (0, 0)),
        ],
        out_specs=[
            pl.BlockSpec((1, 1), lambda i: (0, 0)),
            pl.BlockSpec((1, 1), lambda i: (0, 0)),
        ],
        out_shape=[
            jax.ShapeDtypeStruct((1, 1), jnp.float32),
            jax.ShapeDtypeStruct((1, 1), jnp.float32),
        ],
    )(outputs, t2)
    return (a1.reshape(1), a5.reshape(1))


# speedup vs baseline: 7.4743x; 1.1442x over previous
import functools

import jax
import jax.numpy as jnp
from jax.experimental import pallas as pl

_ROWS_PER_BLOCK = 32
_NCHUNKS = 8


def _acc_kernel(x_ref, t_ref, acc1_ref, acc5_ref, *, scale, nchunks):
    i = pl.program_id(0)
    r, n = x_ref.shape
    cw = n // nchunks
    t = t_ref[pl.ds(i * r, r), :]
    maxes = []
    for c in range(nchunks):
        xc = x_ref[:, c * cw:(c + 1) * cw]
        colc = jax.lax.broadcasted_iota(jnp.int32, (r, cw), 1) + c * cw
        maxes.append(jnp.max(jnp.where(colc == t, xc, -jnp.inf),
                             axis=1, keepdims=True))
    vt = functools.reduce(jnp.maximum, maxes)
    cnts = []
    for c in range(nchunks):
        xc = x_ref[:, c * cw:(c + 1) * cw]
        colc = jax.lax.broadcasted_iota(jnp.int32, (r, cw), 1) + c * cw
        pred = (xc > vt) | ((xc == vt) & (colc < t))
        cnts.append(jnp.sum(pred.astype(jnp.float32), axis=1, keepdims=True))
    rank = functools.reduce(jnp.add, cnts)
    a1 = jnp.sum((rank < 1.0).astype(jnp.float32)).reshape(1, 1) * scale
    a5 = jnp.sum((rank < 5.0).astype(jnp.float32)).reshape(1, 1) * scale

    @pl.when(i == 0)
    def _init():
        acc1_ref[...] = a1
        acc5_ref[...] = a5

    @pl.when(i != 0)
    def _accum():
        acc1_ref[...] += a1
        acc5_ref[...] += a5


@jax.jit
def kernel(outputs, targets):
    b, n = outputs.shape
    r = _ROWS_PER_BLOCK
    t2 = targets.astype(jnp.int32).reshape(b, 1)
    body = functools.partial(_acc_kernel, scale=100.0 / b, nchunks=_NCHUNKS)
    a1, a5 = pl.pallas_call(
        body,
        grid=(b // r,),
        in_specs=[
            pl.BlockSpec((r, n), lambda i: (i, 0)),
            pl.BlockSpec((b, 1), lambda i: (0, 0)),
        ],
        out_specs=[
            pl.BlockSpec((1, 1), lambda i: (0, 0)),
            pl.BlockSpec((1, 1), lambda i: (0, 0)),
        ],
        out_shape=[
            jax.ShapeDtypeStruct((1, 1), jnp.float32),
            jax.ShapeDtypeStruct((1, 1), jnp.float32),
        ],
    )(outputs, t2)
    return (a1.reshape(1), a5.reshape(1))
